# Initial kernel scaffold; baseline (speedup 1.0000x reference)
import jax
import jax.numpy as jnp
from jax.experimental import pallas as pl


def kernel(x, edge_index, batch, prior, emission, transition, contrastive, W_out):
    raise NotImplementedError("write your pallas kernel here")


# speedup vs baseline: 211.9667x; 211.9667x over previous
import functools

import jax
import jax.numpy as jnp
from jax import lax
from jax.experimental import pallas as pl
from jax.experimental.pallas import tpu as pltpu
from jax.experimental.pallas import tpu_sc as plsc

EPS = 1e-12
G = 8
C = 8
M = 16
NG = 512
HALF = 32
BN = 2000


def _gsum():
    r = lax.broadcasted_iota(jnp.int32, (HALF, 4), 0)
    g = lax.broadcasted_iota(jnp.int32, (HALF, 4), 1)
    return (r // C == g).astype(jnp.float32)


def _gbcast():
    g = lax.broadcasted_iota(jnp.int32, (4, HALF), 0)
    r = lax.broadcasted_iota(jnp.int32, (4, HALF), 1)
    return (r // C == g).astype(jnp.float32)


def _onehot(v, width):
    i = lax.broadcasted_iota(jnp.int32, (v.shape[0], width), 1)
    return (i == v[:, None]).astype(jnp.float32)


def _layer0_body(x_ref, em0_ref, post_ref, lik_ref):
    oh = _onehot(x_ref[0, 0, :], M)
    joint = jnp.dot(oh, em0_ref[0], preferred_element_type=jnp.float32,
                  precision=lax.Precision.HIGHEST)
    den = jnp.dot(joint, _gsum(), preferred_element_type=jnp.float32,
                  precision=lax.Precision.HIGHEST)
    lik_ref[...] = jnp.log(den + EPS)
    denb = jnp.dot(den, _gbcast(), preferred_element_type=jnp.float32,
                  precision=lax.Precision.HIGHEST)
    post_ref[...] = joint / (denb + EPS)


def _layer_body(agg_ref, x_ref, em_ref, td_ref, post_ref, lik_ref):
    a = agg_ref[...]
    S = _gsum()
    ST = _gbcast()
    s = jnp.dot(a, S, preferred_element_type=jnp.float32,
                  precision=lax.Precision.HIGHEST)
    sb = jnp.dot(s, ST, preferred_element_type=jnp.float32,
                  precision=lax.Precision.HIGHEST)
    nrm = jnp.where(sb > 0, a / (sb + EPS), 1.0 / C)
    trans = jnp.dot(nrm, td_ref[0], preferred_element_type=jnp.float32)
    oh = _onehot(x_ref[0, 0, :], M)
    em = jnp.dot(oh, em_ref[0], preferred_element_type=jnp.float32,
                  precision=lax.Precision.HIGHEST)
    joint = em * trans
    den = jnp.dot(joint, S, preferred_element_type=jnp.float32,
                  precision=lax.Precision.HIGHEST)
    lik_ref[...] = jnp.log(den + EPS)
    denb = jnp.dot(den, ST, preferred_element_type=jnp.float32,
                  precision=lax.Precision.HIGHEST)
    post_ref[...] = joint / (denb + EPS)


def _make_final(nb, bn, lcols):
    def body(lik_ref, b_ref, ct_ref, wt_ref, out_ref, gl_scr):
        i = pl.program_id(0)
        oh = _onehot(b_ref[0, 0, :], NG)
        contrib = lax.dot_general(
            oh, lik_ref[...], (((0,), (0,)), ((), ())),
            preferred_element_type=jnp.float32,
                  precision=lax.Precision.HIGHEST)

        @pl.when(i == 0)
        def _():
            gl_scr[...] = contrib

        @pl.when(i > 0)
        def _():
            gl_scr[...] += contrib

        @pl.when(i == nb - 1)
        def _():
            act = jnp.tanh(jnp.dot(gl_scr[...], ct_ref[...],
                                   preferred_element_type=jnp.float32))
            out_ref[...] = jnp.dot(act, wt_ref[...],
                                   preferred_element_type=jnp.float32)
    return body


def _make_edge_kernel(N, E):
    SUP = 640
    J = SUP // 128
    NSUP = E // SUP
    NT = 16
    ITERS = (NSUP + NT - 1) // NT
    SLAB = -(-(N // NT) // 8) * 8
    LAST = N - (NT - 1) * SLAB
    NPAD = NT * SLAB
    ZR = 136
    ZK = SLAB // ZR
    assert E % SUP == 0 and SLAB % ZR == 0 and LAST % 8 == 0 and LAST > 0
    assert ZR <= SUP

    mesh = plsc.VectorSubcoreMesh(core_axis_name="c", subcore_axis_name="s")

    @functools.partial(
        pl.kernel, mesh=mesh,
        compiler_params=pltpu.CompilerParams(use_tc_tiling_on_sc=False),
        out_type=jax.ShapeDtypeStruct((2 * N, HALF), jnp.float32),
        scratch_types=[
            pltpu.VMEM((SUP,), jnp.int32),
            pltpu.VMEM((J, 128), jnp.int32),
            pltpu.VMEM((SUP, HALF), jnp.float32),
            pltpu.VMEM_SHARED((NPAD, HALF), jnp.float32),
            pltpu.SemaphoreType.DMA,
            pltpu.SemaphoreType.DMA,
        ],
    )
    def edge_kernel(edges, post, agg_out, sidx, dst2d, rows, agg_sh,
                    sem, sem2):
        c = lax.axis_index("c")
        s = lax.axis_index("s")
        base = c * N

        def zb(r, carry):
            rows[r, pl.ds(0, 16)] = jnp.zeros((16,), jnp.float32)
            rows[r, pl.ds(16, 16)] = jnp.zeros((16,), jnp.float32)
            return carry
        lax.fori_loop(0, ZR, zb, None)
        for k in range(ZK):
            pltpu.sync_copy(rows.at[pl.ds(0, ZR)],
                            agg_sh.at[pl.ds(s * SLAB + k * ZR, ZR)])
        plsc.subcore_barrier()

        def super_body(i, carry):
            sc_id = i * NT + s

            @pl.when(sc_id < NSUP)
            def _():
                off = sc_id * SUP
                cp0 = pltpu.async_copy(edges.at[0, pl.ds(off, SUP)], sidx, sem)
                cps = [pltpu.async_copy(edges.at[1, pl.ds(off + j * 128, 128)],
                                        dst2d.at[j], sem) for j in range(J)]
                cp0.wait()
                for cp in cps:
                    cp.wait()

                def gix(k, carry2):
                    o = pl.multiple_of(k * 16, 16)
                    sidx[pl.ds(o, 16)] = sidx[pl.ds(o, 16)] + base
                    return carry2
                lax.fori_loop(0, SUP // 16, gix, None)

                gcs = [pltpu.async_copy(post.at[sidx.at[pl.ds(j * 128, 128)]],
                                        rows.at[pl.ds(j * 128, 128), :], sem)
                       for j in range(J)]
                for cp in gcs:
                    cp.wait()
                scs = [pltpu.async_copy(rows.at[pl.ds(j * 128, 128), :],
                                        agg_sh.at[dst2d.at[j]], sem2, add=True)
                       for j in range(J)]
                for cp in scs:
                    cp.wait()
            return carry
        lax.fori_loop(0, ITERS, super_body, None)

        plsc.subcore_barrier()
        o = s * SLAB

        @pl.when(s < NT - 1)
        def _():
            pltpu.sync_copy(agg_sh.at[pl.ds(o, SLAB)],
                            agg_out.at[pl.ds(base + o, SLAB)])

        @pl.when(s == NT - 1)
        def _():
            pltpu.sync_copy(agg_sh.at[pl.ds(o, LAST)],
                            agg_out.at[pl.ds(base + o, LAST)])

    return edge_kernel


def kernel(x, edge_index, batch, prior, emission, transition, contrastive,
           W_out):
    N = x.shape[0]
    E = edge_index.shape[1]
    L = emission.shape[0]
    CU = contrastive.shape[1]
    NB = N // BN
    assert N % BN == 0

    x = x.astype(jnp.int32)
    batch = batch.astype(jnp.int32)
    edge_index = edge_index.astype(jnp.int32)

    emt = jnp.transpose(emission, (0, 3, 1, 2)).reshape(L, M, G * C)
    em0p = emt[0] * prior.reshape(1, G * C)
    em_tabs = [jnp.stack([t[:, :HALF], t[:, HALF:]])
               for t in [em0p] + [emt[l] for l in range(1, L)]]
    blocks = jnp.transpose(transition, (0, 1, 3, 2))
    eye8 = jnp.eye(G, dtype=jnp.float32)
    td64 = (eye8[None, :, None, :, None]
            * blocks[:, :, :, None, :]).reshape(L, G * C, G * C)
    td_tabs = [jnp.stack([td64[l, :HALF, :HALF], td64[l, HALF:, HALF:]])
               for l in range(L)]
    ct4 = jnp.kron(jnp.eye(L, dtype=jnp.float32), contrastive)
    wt = W_out.T
    x3 = x.reshape(NB, 1, BN)

    node_grid = (2, NB)
    x_spec = pl.BlockSpec((1, 1, BN), lambda h, i: (i, 0, 0))
    tab_spec = lambda r: pl.BlockSpec((1, r, HALF), lambda h, i: (h, 0, 0))
    row_spec = pl.BlockSpec((BN, HALF), lambda h, i: (h * NB + i, 0))
    lik_spec = pl.BlockSpec((BN, 4), lambda h, i: (h * NB + i, 0))

    post, lik0 = pl.pallas_call(
        _layer0_body,
        grid=node_grid,
        in_specs=[x_spec, tab_spec(M)],
        out_specs=[row_spec, lik_spec],
        out_shape=[jax.ShapeDtypeStruct((2 * N, HALF), jnp.float32),
                   jax.ShapeDtypeStruct((2 * N, 4), jnp.float32)],
    )(x3, em_tabs[0])

    edge_kernel = _make_edge_kernel(N, E)
    liks = [lik0]
    for l in range(1, L):
        agg = edge_kernel(edge_index, post)
        post, likl = pl.pallas_call(
            _layer_body,
            grid=node_grid,
            in_specs=[row_spec, x_spec, tab_spec(M), tab_spec(HALF)],
            out_specs=[row_spec, lik_spec],
            out_shape=[jax.ShapeDtypeStruct((2 * N, HALF), jnp.float32),
                       jax.ShapeDtypeStruct((2 * N, 4), jnp.float32)],
        )(agg, x3, em_tabs[l], td_tabs[l])
        liks.append(likl)

    lik_all = jnp.concatenate(
        [jnp.concatenate([lk[:N], lk[N:]], axis=-1) for lk in liks], axis=-1)

    lcols = L * G
    b3 = batch.reshape(NB, 1, BN)
    out = pl.pallas_call(
        _make_final(NB, BN, lcols),
        grid=(NB,),
        in_specs=[pl.BlockSpec((BN, lcols), lambda i: (i, 0)),
                  pl.BlockSpec((1, 1, BN), lambda i: (i, 0, 0)),
                  pl.BlockSpec((lcols, L * CU), lambda i: (0, 0)),
                  pl.BlockSpec((L * CU, W_out.shape[0]), lambda i: (0, 0))],
        out_specs=pl.BlockSpec((NG, W_out.shape[0]), lambda i: (0, 0)),
        out_shape=jax.ShapeDtypeStruct((NG, W_out.shape[0]), jnp.float32),
        scratch_shapes=[pltpu.VMEM((NG, lcols), jnp.float32)],
    )(lik_all, b3, ct4, wt)
    return out
